# Initial kernel scaffold; baseline (speedup 1.0000x reference)
import jax
import jax.numpy as jnp
from jax.experimental import pallas as pl


def kernel(entity_emb, user_emb, latent_emb, edge_index, edge_type, interact_rows, interact_cols, interact_vals, weight, disen_weight_att):
    raise NotImplementedError("write your pallas kernel here")


# speedup vs baseline: 3.2633x; 3.2633x over previous
import functools

import jax
import jax.numpy as jnp
from jax import lax
from jax.experimental import pallas as pl
from jax.experimental.pallas import tpu as pltpu
from jax.experimental.pallas import tpu_sc as plsc

CH = 128
LANES = 16
BLK = 64
NSUB = 16
NCORE = 2
ACC_ROWS = 10240
GARBAGE = 10000


def _sc_body(ent_hbm, w_hbm, kg_tail, kg_head, kg_type, u_cols, u_rows,
             u_vals, out_ent, out_usr,
             acc, cnt, idx_a, idx_b, idx_c, valbuf, rows_v, rel_v, ones_v,
             cb_v, zcnt_v, sem, *, nb1, nb2):
    cid = lax.axis_index("c")
    sid = lax.axis_index("s")
    NB1 = nb1
    NB2 = nb2

    rows_per_tile = ACC_ROWS // NSUB

    def zero_row(r, _):
        zf = jnp.zeros((LANES,), jnp.float32)
        for k in range(CH // LANES):
            rows_v[r, pl.ds(k * LANES, LANES)] = zf
        return 0

    lax.fori_loop(0, BLK, zero_row, 0)

    def zero_cnt1(g, _):
        zcnt_v[pl.ds(g * LANES, LANES)] = jnp.zeros((LANES,), jnp.float32)
        return 0

    lax.fori_loop(0, rows_per_tile // LANES, zero_cnt1, 0)

    def fill_ones(g, _):
        ones_v[pl.ds(g * LANES, LANES)] = jnp.full((LANES,), 1.0, jnp.float32)
        return 0

    lax.fori_loop(0, BLK // LANES, fill_ones, 0)

    for j in range(rows_per_tile // BLK):
        r0 = sid * rows_per_tile + j * BLK
        pltpu.sync_copy(rows_v, acc.at[pl.ds(r0, BLK)])
    pltpu.sync_copy(zcnt_v, cnt.at[pl.ds(sid * rows_per_tile, rows_per_tile)])
    plsc.subcore_barrier()

    @pl.when(cid == 0)
    def _kg():
        def block(b, _):
            base = (sid * NB1 + b) * BLK
            pltpu.sync_copy(kg_tail.at[pl.ds(base, BLK)], idx_a)
            pltpu.sync_copy(kg_head.at[pl.ds(base, BLK)], idx_b)
            pltpu.sync_copy(kg_type.at[pl.ds(base, BLK)], idx_c)
            for c in range(BLK // LANES):
                sl = pl.ds(c * LANES, LANES)
                idx_c[sl] = idx_c[sl] - 1
            pltpu.async_copy(ent_hbm.at[idx_a], rows_v, sem).wait()
            pltpu.async_copy(w_hbm.at[idx_c], rel_v, sem).wait()

            def mul_row(r, _):
                for k in range(CH // LANES):
                    sl = pl.ds(k * LANES, LANES)
                    rows_v[r, sl] = rows_v[r, sl] * rel_v[r, sl]
                return 0

            lax.fori_loop(0, BLK, mul_row, 0)
            pltpu.sync_copy(rows_v, acc.at[idx_b], add=True)
            pltpu.sync_copy(ones_v, cnt.at[idx_b], add=True)
            return 0

        lax.fori_loop(0, NB1, block, 0)

    @pl.when(cid == 1)
    def _usr():
        def block(b, _):
            base = (sid * NB2 + b) * BLK
            pltpu.sync_copy(u_cols.at[pl.ds(base, BLK)], idx_a)
            pltpu.sync_copy(u_rows.at[pl.ds(base, BLK)], idx_b)
            pltpu.sync_copy(u_vals.at[pl.ds(base, BLK)], valbuf)
            pltpu.async_copy(ent_hbm.at[idx_a], rows_v, sem).wait()

            def mul_group(g, _):
                vv = valbuf[pl.ds(g * LANES, LANES)]
                for j in range(LANES):
                    e = g * LANES + j
                    vj = vv[j]
                    for k in range(CH // LANES):
                        sl = pl.ds(k * LANES, LANES)
                        rows_v[e, sl] = rows_v[e, sl] * vj
                return 0

            lax.fori_loop(0, BLK // LANES, mul_group, 0)
            pltpu.sync_copy(rows_v, acc.at[idx_b], add=True)
            return 0

        lax.fori_loop(0, NB2, block, 0)

    plsc.subcore_barrier()

    rpt = ACC_ROWS // NSUB

    @pl.when(cid == 0)
    def _wb_ent():
        def wchunk(j, _):
            r0 = sid * rpt + j * BLK
            pltpu.sync_copy(acc.at[pl.ds(r0, BLK)], rows_v)
            pltpu.sync_copy(cnt.at[pl.ds(r0, BLK)], cb_v)

            def fix_group(g, _):
                cvec = cb_v[pl.ds(g * LANES, LANES)]
                inv = 1.0 / jnp.maximum(cvec, 1.0)
                for j2 in range(LANES):
                    r = g * LANES + j2
                    ij = inv[j2]
                    for k in range(CH // LANES):
                        sl = pl.ds(k * LANES, LANES)
                        rows_v[r, sl] = rows_v[r, sl] * ij
                return 0

            lax.fori_loop(0, BLK // LANES, fix_group, 0)
            pltpu.sync_copy(rows_v, out_ent.at[pl.ds(r0, BLK)])
            return 0

        lax.fori_loop(0, rpt // BLK, wchunk, 0)

    @pl.when(cid == 1)
    def _wb_usr():
        def wchunk(j, _):
            r0 = sid * rpt + j * BLK
            pltpu.sync_copy(acc.at[pl.ds(r0, BLK)], rows_v)
            pltpu.sync_copy(rows_v, out_usr.at[pl.ds(r0, BLK)])
            return 0

        lax.fori_loop(0, rpt // BLK, wchunk, 0)


def _sc_aggregate(entity_emb, weight, kg_tail, kg_head, kg_type,
                  u_cols, u_rows, u_vals, nb1, nb2):
    mesh = plsc.VectorSubcoreMesh(core_axis_name="c", subcore_axis_name="s",
                                  num_cores=NCORE, num_subcores=NSUB)
    body = functools.partial(_sc_body, nb1=nb1, nb2=nb2)
    f = pl.kernel(
        body,
        out_type=(
            jax.ShapeDtypeStruct((ACC_ROWS, CH), jnp.float32),
            jax.ShapeDtypeStruct((ACC_ROWS, CH), jnp.float32),
        ),
        mesh=mesh,
        scratch_types=[
            pltpu.VMEM_SHARED((ACC_ROWS, CH), jnp.float32),
            pltpu.VMEM_SHARED((ACC_ROWS,), jnp.float32),
            pltpu.VMEM((BLK,), jnp.int32),
            pltpu.VMEM((BLK,), jnp.int32),
            pltpu.VMEM((BLK,), jnp.int32),
            pltpu.VMEM((BLK,), jnp.float32),
            pltpu.VMEM((BLK, CH), jnp.float32),
            pltpu.VMEM((BLK, CH), jnp.float32),
            pltpu.VMEM((BLK,), jnp.float32),
            pltpu.VMEM((BLK,), jnp.float32),
            pltpu.VMEM((ACC_ROWS // NSUB,), jnp.float32),
            pltpu.SemaphoreType.DMA,
        ],
    )
    return f(entity_emb, weight, kg_tail, kg_head, kg_type,
             u_cols, u_rows, u_vals)


def _tc_body(ue_ref, lat_ref, dis_ref, w_ref, ua_ref, out_ref):
    ue = ue_ref[...]
    s = lax.dot_general(ue, lat_ref[...],
                        (((1,), (1,)), ((), ())))
    s = jax.nn.softmax(s, axis=1)
    d = jax.nn.softmax(dis_ref[...], axis=-1) @ w_ref[...]
    m = 1.0 + s @ d
    out_ref[...] = ua_ref[...] * m


def _tc_modulate(user_emb, latent_emb, disen_weight_att, weight, user_agg):
    n_usr = user_emb.shape[0]
    BU = 1000
    grid = (n_usr // BU,)
    return pl.pallas_call(
        _tc_body,
        grid=grid,
        in_specs=[
            pl.BlockSpec((BU, CH), lambda i: (i, 0)),
            pl.BlockSpec(latent_emb.shape, lambda i: (0, 0)),
            pl.BlockSpec(disen_weight_att.shape, lambda i: (0, 0)),
            pl.BlockSpec(weight.shape, lambda i: (0, 0)),
            pl.BlockSpec((BU, CH), lambda i: (i, 0)),
        ],
        out_specs=pl.BlockSpec((BU, CH), lambda i: (i, 0)),
        out_shape=jax.ShapeDtypeStruct((n_usr, CH), jnp.float32),
    )(user_emb, latent_emb, disen_weight_att, weight, user_agg)


def _pad_to(x, n, fill):
    pad = n - x.shape[0]
    return jnp.concatenate([x, jnp.full((pad,), fill, x.dtype)])


def kernel(entity_emb, user_emb, latent_emb, edge_index, edge_type,
           interact_rows, interact_cols, interact_vals, weight,
           disen_weight_att):
    head = edge_index[0].astype(jnp.int32)
    tail = edge_index[1].astype(jnp.int32)
    et = edge_type.astype(jnp.int32)
    ur = interact_rows.astype(jnp.int32)
    uc = interact_cols.astype(jnp.int32)

    unit = BLK * NSUB
    e1 = head.shape[0]
    e2 = ur.shape[0]
    nb1 = -(-e1 // unit)
    nb2 = -(-e2 // unit)
    e1p = nb1 * unit
    e2p = nb2 * unit

    kg_tail = _pad_to(tail, e1p, 0)
    kg_head = _pad_to(head, e1p, GARBAGE)
    kg_type = _pad_to(et, e1p, 1)
    u_cols = _pad_to(uc, e2p, 0)
    u_rows = _pad_to(ur, e2p, GARBAGE)
    u_vals = _pad_to(interact_vals, e2p, 0.0)

    entity_agg, user_agg = _sc_aggregate(
        entity_emb, weight, kg_tail, kg_head, kg_type,
        u_cols, u_rows, u_vals, nb1, nb2)
    n_ent = entity_emb.shape[0]
    entity_agg = entity_agg[:n_ent]
    user_agg = user_agg[:n_ent]

    user_out = _tc_modulate(user_emb, latent_emb, disen_weight_att, weight,
                            user_agg)
    return (entity_agg, user_out)


# speedup vs baseline: 3.8979x; 1.1945x over previous
import functools

import jax
import jax.numpy as jnp
from jax import lax
from jax.experimental import pallas as pl
from jax.experimental.pallas import tpu as pltpu
from jax.experimental.pallas import tpu_sc as plsc

CH = 128
LANES = 16
BLK = 64
PK = 3 * BLK
NSUB = 16
NCORE = 2
ACC_ROWS = 10240
GARBAGE = 10000


def _sc_body(ent_hbm, w_hbm, kg_pack, u_pack, u_vals, out_ent, out_usr,
             acc, cnt, ip0, ip1, sb0, sb1, rw0, rw1, rl0, rl1,
             vb0, vb1, ones_v, cb_v, zcnt_v, sg0, sg1, *, nb1, nb2):
    cid = lax.axis_index("c")
    sid = lax.axis_index("s")
    NB1 = nb1
    NB2 = nb2

    rows_per_tile = ACC_ROWS // NSUB

    def zero_row(r, _):
        zf = jnp.zeros((LANES,), jnp.float32)
        for k in range(CH // LANES):
            rw0[r, pl.ds(k * LANES, LANES)] = zf
        return 0

    lax.fori_loop(0, BLK, zero_row, 0)

    def zero_cnt1(g, _):
        zcnt_v[pl.ds(g * LANES, LANES)] = jnp.zeros((LANES,), jnp.float32)
        return 0

    lax.fori_loop(0, rows_per_tile // LANES, zero_cnt1, 0)

    def fill_ones(g, _):
        ones_v[pl.ds(g * LANES, LANES)] = jnp.full((LANES,), 1.0, jnp.float32)
        return 0

    lax.fori_loop(0, BLK // LANES, fill_ones, 0)

    for j in range(rows_per_tile // BLK):
        r0 = sid * rows_per_tile + j * BLK
        pltpu.sync_copy(rw0, acc.at[pl.ds(r0, BLK)])
    pltpu.sync_copy(zcnt_v, cnt.at[pl.ds(sid * rows_per_tile, rows_per_tile)])
    plsc.subcore_barrier()

    @pl.when(cid == 0)
    def _kg():
        def issue(b, ip, sb, rw, rl, sg):
            base = (sid * NB1 + b) * PK
            pltpu.sync_copy(kg_pack.at[pl.ds(base, PK)], ip)
            for k in range(BLK // LANES):
                sb[pl.ds(k * LANES, LANES)] = ip[pl.ds(BLK + k * LANES, LANES)]
            pltpu.async_copy(ent_hbm.at[ip.at[pl.ds(0, BLK)]], rw, sg)
            pltpu.async_copy(w_hbm.at[ip.at[pl.ds(2 * BLK, BLK)]], rl, sg)

        def drain(ip, rw, rl, sg):
            pltpu.make_async_copy(ent_hbm.at[ip.at[pl.ds(0, BLK)]], rw,
                                  sg).wait()
            pltpu.make_async_copy(w_hbm.at[ip.at[pl.ds(2 * BLK, BLK)]], rl,
                                  sg).wait()

        def finish(ip, sb, rw, rl, sg):
            drain(ip, rw, rl, sg)

            def mul_row(r2, _):
                for u in range(2):
                    r = 2 * r2 + u
                    for k in range(CH // LANES):
                        sl = pl.ds(k * LANES, LANES)
                        rw[r, sl] = rw[r, sl] * rl[r, sl]
                return 0

            lax.fori_loop(0, BLK // 2, mul_row, 0)
            pltpu.sync_copy(rw, acc.at[sb], add=True)
            pltpu.sync_copy(ones_v, cnt.at[sb], add=True)

        issue(0, ip0, sb0, rw0, rl0, sg0)

        def pair(h, _):
            b0 = 2 * h
            issue(b0 + 1, ip1, sb1, rw1, rl1, sg1)
            finish(ip0, sb0, rw0, rl0, sg0)
            issue(b0 + 2, ip0, sb0, rw0, rl0, sg0)
            finish(ip1, sb1, rw1, rl1, sg1)
            return 0

        lax.fori_loop(0, NB1 // 2, pair, 0)
        drain(ip0, rw0, rl0, sg0)

    @pl.when(cid == 1)
    def _usr():
        def issue(b, ip, sb, rw, vb, sg):
            base = (sid * NB2 + b) * (2 * BLK)
            pltpu.sync_copy(u_pack.at[pl.ds(base, 2 * BLK)],
                            ip.at[pl.ds(0, 2 * BLK)])
            vbase = (sid * NB2 + b) * BLK
            pltpu.sync_copy(u_vals.at[pl.ds(vbase, BLK)], vb)
            for k in range(BLK // LANES):
                sb[pl.ds(k * LANES, LANES)] = ip[pl.ds(BLK + k * LANES, LANES)]
            pltpu.async_copy(ent_hbm.at[ip.at[pl.ds(0, BLK)]], rw, sg)

        def drain(ip, rw, sg):
            pltpu.make_async_copy(ent_hbm.at[ip.at[pl.ds(0, BLK)]], rw,
                                  sg).wait()

        def finish(ip, sb, rw, vb, sg):
            drain(ip, rw, sg)

            def mul_group(g, _):
                vv = vb[pl.ds(g * LANES, LANES)]
                for j in range(LANES):
                    e = g * LANES + j
                    vj = vv[j]
                    for k in range(CH // LANES):
                        sl = pl.ds(k * LANES, LANES)
                        rw[e, sl] = rw[e, sl] * vj
                return 0

            lax.fori_loop(0, BLK // LANES, mul_group, 0)
            pltpu.sync_copy(rw, acc.at[sb], add=True)

        issue(0, ip0, sb0, rw0, vb0, sg0)

        def pair(h, _):
            b0 = 2 * h
            issue(b0 + 1, ip1, sb1, rw1, vb1, sg1)
            finish(ip0, sb0, rw0, vb0, sg0)
            issue(b0 + 2, ip0, sb0, rw0, vb0, sg0)
            finish(ip1, sb1, rw1, vb1, sg1)
            return 0

        lax.fori_loop(0, NB2 // 2, pair, 0)
        drain(ip0, rw0, sg0)

    plsc.subcore_barrier()

    rpt = ACC_ROWS // NSUB

    @pl.when(cid == 0)
    def _wb_ent():
        def wchunk(j, _):
            r0 = sid * rpt + j * BLK
            pltpu.sync_copy(acc.at[pl.ds(r0, BLK)], rw0)
            pltpu.sync_copy(cnt.at[pl.ds(r0, BLK)], cb_v)

            def fix_group(g, _):
                cvec = cb_v[pl.ds(g * LANES, LANES)]
                inv = 1.0 / jnp.maximum(cvec, 1.0)
                for j2 in range(LANES):
                    r = g * LANES + j2
                    ij = inv[j2]
                    for k in range(CH // LANES):
                        sl = pl.ds(k * LANES, LANES)
                        rw0[r, sl] = rw0[r, sl] * ij
                return 0

            lax.fori_loop(0, BLK // LANES, fix_group, 0)
            pltpu.sync_copy(rw0, out_ent.at[pl.ds(r0, BLK)])
            return 0

        lax.fori_loop(0, rpt // BLK, wchunk, 0)

    @pl.when(cid == 1)
    def _wb_usr():
        def wchunk(j, _):
            r0 = sid * rpt + j * BLK
            pltpu.sync_copy(acc.at[pl.ds(r0, BLK)], rw0)
            pltpu.sync_copy(rw0, out_usr.at[pl.ds(r0, BLK)])
            return 0

        lax.fori_loop(0, rpt // BLK, wchunk, 0)


def _sc_aggregate(entity_emb, weight, kg_pack, u_pack, u_vals, nb1, nb2):
    mesh = plsc.VectorSubcoreMesh(core_axis_name="c", subcore_axis_name="s",
                                  num_cores=NCORE, num_subcores=NSUB)
    body = functools.partial(_sc_body, nb1=nb1, nb2=nb2)
    f = pl.kernel(
        body,
        out_type=(
            jax.ShapeDtypeStruct((ACC_ROWS, CH), jnp.float32),
            jax.ShapeDtypeStruct((ACC_ROWS, CH), jnp.float32),
        ),
        mesh=mesh,
        scratch_types=[
            pltpu.VMEM_SHARED((ACC_ROWS, CH), jnp.float32),
            pltpu.VMEM_SHARED((ACC_ROWS,), jnp.float32),
            pltpu.VMEM((PK,), jnp.int32),
            pltpu.VMEM((PK,), jnp.int32),
            pltpu.VMEM((BLK,), jnp.int32),
            pltpu.VMEM((BLK,), jnp.int32),
            pltpu.VMEM((BLK, CH), jnp.float32),
            pltpu.VMEM((BLK, CH), jnp.float32),
            pltpu.VMEM((BLK, CH), jnp.float32),
            pltpu.VMEM((BLK, CH), jnp.float32),
            pltpu.VMEM((BLK,), jnp.float32),
            pltpu.VMEM((BLK,), jnp.float32),
            pltpu.VMEM((BLK,), jnp.float32),
            pltpu.VMEM((BLK,), jnp.float32),
            pltpu.VMEM((ACC_ROWS // NSUB,), jnp.float32),
            pltpu.SemaphoreType.DMA,
            pltpu.SemaphoreType.DMA,
        ],
    )
    return f(entity_emb, weight, kg_pack, u_pack, u_vals)


def _tc_body(ue_ref, lat_ref, dis_ref, w_ref, ua_ref, out_ref):
    ue = ue_ref[...]
    s = lax.dot_general(ue, lat_ref[...],
                        (((1,), (1,)), ((), ())))
    s = jax.nn.softmax(s, axis=1)
    d = jax.nn.softmax(dis_ref[...], axis=-1) @ w_ref[...]
    m = 1.0 + s @ d
    out_ref[...] = ua_ref[...] * m


def _tc_modulate(user_emb, latent_emb, disen_weight_att, weight, user_agg):
    n_usr = user_emb.shape[0]
    BU = 1000
    grid = (n_usr // BU,)
    return pl.pallas_call(
        _tc_body,
        grid=grid,
        in_specs=[
            pl.BlockSpec((BU, CH), lambda i: (i, 0)),
            pl.BlockSpec(latent_emb.shape, lambda i: (0, 0)),
            pl.BlockSpec(disen_weight_att.shape, lambda i: (0, 0)),
            pl.BlockSpec(weight.shape, lambda i: (0, 0)),
            pl.BlockSpec((BU, CH), lambda i: (i, 0)),
        ],
        out_specs=pl.BlockSpec((BU, CH), lambda i: (i, 0)),
        out_shape=jax.ShapeDtypeStruct((n_usr, CH), jnp.float32),
    )(user_emb, latent_emb, disen_weight_att, weight, user_agg)


def _packn(arrs, total_blocks):
    n = total_blocks
    m = jnp.stack([a.reshape(n, BLK) for a in arrs], axis=1)
    return m.reshape(-1)


def _pad_to(x, n, fill):
    pad = n - x.shape[0]
    return jnp.concatenate([x, jnp.full((pad,), fill, x.dtype)])


def kernel(entity_emb, user_emb, latent_emb, edge_index, edge_type,
           interact_rows, interact_cols, interact_vals, weight,
           disen_weight_att):
    head = edge_index[0].astype(jnp.int32)
    tail = edge_index[1].astype(jnp.int32)
    et = edge_type.astype(jnp.int32)
    ur = interact_rows.astype(jnp.int32)
    uc = interact_cols.astype(jnp.int32)

    unit = BLK * NSUB
    e1 = head.shape[0]
    e2 = ur.shape[0]
    nb1 = -(-e1 // unit)
    nb1 += nb1 % 2
    nb2 = -(-e2 // unit)
    nb2 += nb2 % 2
    tb1 = nb1 * NSUB + 1
    tb2 = nb2 * NSUB + 1

    kg_pack = _packn([
        _pad_to(tail, tb1 * BLK, 0),
        _pad_to(head, tb1 * BLK, GARBAGE),
        _pad_to(et, tb1 * BLK, 1) - 1,
    ], tb1)
    u_pack = _packn([
        _pad_to(uc, tb2 * BLK, 0),
        _pad_to(ur, tb2 * BLK, GARBAGE),
    ], tb2)
    u_vals = _pad_to(interact_vals, tb2 * BLK, 0.0)

    entity_agg, user_agg = _sc_aggregate(
        entity_emb, weight, kg_pack, u_pack, u_vals, nb1, nb2)
    n_ent = entity_emb.shape[0]
    entity_agg = entity_agg[:n_ent]
    user_agg = user_agg[:n_ent]

    user_out = _tc_modulate(user_emb, latent_emb, disen_weight_att, weight,
                            user_agg)
    return (entity_agg, user_out)


# speedup vs baseline: 4.5545x; 1.1684x over previous
import functools

import jax
import jax.numpy as jnp
from jax import lax
from jax.experimental import pallas as pl
from jax.experimental.pallas import tpu as pltpu
from jax.experimental.pallas import tpu_sc as plsc

CH = 128
LANES = 16
BLK = 128
PK = 3 * BLK
NSUB = 16
NCORE = 2
ACC_ROWS = 10240
GARBAGE = 10000


def _sc_body(ent_hbm, w_hbm, kg_pack, u_pack, u_vals, out_ent, out_usr,
             acc, cnt, ip0, ip1, sb0, sb1, rw0, rw1, wtab,
             vb0, vb1, ones_v, cb_v, zcnt_v, sg0, sg1, *, nb1, nb2):
    cid = lax.axis_index("c")
    sid = lax.axis_index("s")
    NB1 = nb1
    NB2 = nb2

    rows_per_tile = ACC_ROWS // NSUB

    def zero_row(r, _):
        zf = jnp.zeros((LANES,), jnp.float32)
        for k in range(CH // LANES):
            rw0[r, pl.ds(k * LANES, LANES)] = zf
        return 0

    lax.fori_loop(0, BLK, zero_row, 0)

    def zero_cnt1(g, _):
        zcnt_v[pl.ds(g * LANES, LANES)] = jnp.zeros((LANES,), jnp.float32)
        return 0

    lax.fori_loop(0, rows_per_tile // LANES, zero_cnt1, 0)

    def fill_ones(g, _):
        ones_v[pl.ds(g * LANES, LANES)] = jnp.full((LANES,), 1.0, jnp.float32)
        return 0

    lax.fori_loop(0, BLK // LANES, fill_ones, 0)

    for j in range(rows_per_tile // BLK):
        r0 = sid * rows_per_tile + j * BLK
        pltpu.sync_copy(rw0, acc.at[pl.ds(r0, BLK)])
    pltpu.sync_copy(zcnt_v, cnt.at[pl.ds(sid * rows_per_tile, rows_per_tile)])
    plsc.subcore_barrier()

    @pl.when(cid == 0)
    def _kg():
        pltpu.sync_copy(w_hbm, wtab)

        def issue(b, ip, sb, rw, sg):
            base = (sid * NB1 + b) * PK
            pltpu.sync_copy(kg_pack.at[pl.ds(base, PK)], ip)
            for k in range(BLK // LANES):
                sb[pl.ds(k * LANES, LANES)] = ip[pl.ds(BLK + k * LANES, LANES)]
            pltpu.async_copy(ent_hbm.at[ip.at[pl.ds(0, BLK)]], rw, sg)

        def drain(ip, rw, sg):
            pltpu.make_async_copy(ent_hbm.at[ip.at[pl.ds(0, BLK)]], rw,
                                  sg).wait()

        def finish(ip, sb, rw, sg):
            drain(ip, rw, sg)

            def mul_group(g, _):
                tv = ip[pl.ds(2 * BLK + g * LANES, LANES)]
                for j in range(LANES):
                    e = g * LANES + j
                    t = tv[j]
                    for k in range(CH // LANES):
                        sl = pl.ds(k * LANES, LANES)
                        rw[e, sl] = rw[e, sl] * wtab[t, sl]
                return 0

            lax.fori_loop(0, BLK // LANES, mul_group, 0)
            pltpu.sync_copy(rw, acc.at[sb], add=True)
            pltpu.sync_copy(ones_v, cnt.at[sb], add=True)

        issue(0, ip0, sb0, rw0, sg0)

        def pair(h, _):
            b0 = 2 * h
            issue(b0 + 1, ip1, sb1, rw1, sg1)
            finish(ip0, sb0, rw0, sg0)
            issue(b0 + 2, ip0, sb0, rw0, sg0)
            finish(ip1, sb1, rw1, sg1)
            return 0

        lax.fori_loop(0, NB1 // 2, pair, 0)
        drain(ip0, rw0, sg0)

    @pl.when(cid == 1)
    def _usr():
        def issue(b, ip, sb, rw, vb, sg):
            base = (sid * NB2 + b) * (2 * BLK)
            pltpu.sync_copy(u_pack.at[pl.ds(base, 2 * BLK)],
                            ip.at[pl.ds(0, 2 * BLK)])
            vbase = (sid * NB2 + b) * BLK
            pltpu.sync_copy(u_vals.at[pl.ds(vbase, BLK)], vb)
            for k in range(BLK // LANES):
                sb[pl.ds(k * LANES, LANES)] = ip[pl.ds(BLK + k * LANES, LANES)]
            pltpu.async_copy(ent_hbm.at[ip.at[pl.ds(0, BLK)]], rw, sg)

        def drain(ip, rw, sg):
            pltpu.make_async_copy(ent_hbm.at[ip.at[pl.ds(0, BLK)]], rw,
                                  sg).wait()

        def finish(ip, sb, rw, vb, sg):
            drain(ip, rw, sg)

            def mul_group(g, _):
                vv = vb[pl.ds(g * LANES, LANES)]
                for j in range(LANES):
                    e = g * LANES + j
                    vj = vv[j]
                    for k in range(CH // LANES):
                        sl = pl.ds(k * LANES, LANES)
                        rw[e, sl] = rw[e, sl] * vj
                return 0

            lax.fori_loop(0, BLK // LANES, mul_group, 0)
            pltpu.sync_copy(rw, acc.at[sb], add=True)

        issue(0, ip0, sb0, rw0, vb0, sg0)

        def pair(h, _):
            b0 = 2 * h
            issue(b0 + 1, ip1, sb1, rw1, vb1, sg1)
            finish(ip0, sb0, rw0, vb0, sg0)
            issue(b0 + 2, ip0, sb0, rw0, vb0, sg0)
            finish(ip1, sb1, rw1, vb1, sg1)
            return 0

        lax.fori_loop(0, NB2 // 2, pair, 0)
        drain(ip0, rw0, sg0)

    plsc.subcore_barrier()

    rpt = ACC_ROWS // NSUB

    @pl.when(cid == 0)
    def _wb_ent():
        def wchunk(j, _):
            r0 = sid * rpt + j * BLK
            pltpu.sync_copy(acc.at[pl.ds(r0, BLK)], rw0)
            pltpu.sync_copy(cnt.at[pl.ds(r0, BLK)], cb_v)

            def fix_group(g, _):
                cvec = cb_v[pl.ds(g * LANES, LANES)]
                inv = 1.0 / jnp.maximum(cvec, 1.0)
                for j2 in range(LANES):
                    r = g * LANES + j2
                    ij = inv[j2]
                    for k in range(CH // LANES):
                        sl = pl.ds(k * LANES, LANES)
                        rw0[r, sl] = rw0[r, sl] * ij
                return 0

            lax.fori_loop(0, BLK // LANES, fix_group, 0)
            pltpu.sync_copy(rw0, out_ent.at[pl.ds(r0, BLK)])
            return 0

        lax.fori_loop(0, rpt // BLK, wchunk, 0)

    @pl.when(cid == 1)
    def _wb_usr():
        def wchunk(j, _):
            r0 = sid * rpt + j * BLK
            pltpu.sync_copy(acc.at[pl.ds(r0, BLK)], rw0)
            pltpu.sync_copy(rw0, out_usr.at[pl.ds(r0, BLK)])
            return 0

        lax.fori_loop(0, rpt // BLK, wchunk, 0)


def _sc_aggregate(entity_emb, weight, kg_pack, u_pack, u_vals, nb1, nb2):
    mesh = plsc.VectorSubcoreMesh(core_axis_name="c", subcore_axis_name="s",
                                  num_cores=NCORE, num_subcores=NSUB)
    body = functools.partial(_sc_body, nb1=nb1, nb2=nb2)
    f = pl.kernel(
        body,
        out_type=(
            jax.ShapeDtypeStruct((ACC_ROWS, CH), jnp.float32),
            jax.ShapeDtypeStruct((ACC_ROWS, CH), jnp.float32),
        ),
        mesh=mesh,
        scratch_types=[
            pltpu.VMEM_SHARED((ACC_ROWS, CH), jnp.float32),
            pltpu.VMEM_SHARED((ACC_ROWS,), jnp.float32),
            pltpu.VMEM((PK,), jnp.int32),
            pltpu.VMEM((PK,), jnp.int32),
            pltpu.VMEM((BLK,), jnp.int32),
            pltpu.VMEM((BLK,), jnp.int32),
            pltpu.VMEM((BLK, CH), jnp.float32),
            pltpu.VMEM((BLK, CH), jnp.float32),
            pltpu.VMEM((16, CH), jnp.float32),
            pltpu.VMEM((BLK,), jnp.float32),
            pltpu.VMEM((BLK,), jnp.float32),
            pltpu.VMEM((BLK,), jnp.float32),
            pltpu.VMEM((BLK,), jnp.float32),
            pltpu.VMEM((ACC_ROWS // NSUB,), jnp.float32),
            pltpu.SemaphoreType.DMA,
            pltpu.SemaphoreType.DMA,
        ],
    )
    return f(entity_emb, weight, kg_pack, u_pack, u_vals)


def _tc_body(ue_ref, lat_ref, dis_ref, w_ref, ua_ref, out_ref):
    ue = ue_ref[...]
    s = lax.dot_general(ue, lat_ref[...],
                        (((1,), (1,)), ((), ())))
    s = jax.nn.softmax(s, axis=1)
    d = jax.nn.softmax(dis_ref[...], axis=-1) @ w_ref[...]
    m = 1.0 + s @ d
    out_ref[...] = ua_ref[...] * m


def _tc_modulate(user_emb, latent_emb, disen_weight_att, weight, user_agg):
    n_usr = user_emb.shape[0]
    BU = 1000
    grid = (n_usr // BU,)
    return pl.pallas_call(
        _tc_body,
        grid=grid,
        in_specs=[
            pl.BlockSpec((BU, CH), lambda i: (i, 0)),
            pl.BlockSpec(latent_emb.shape, lambda i: (0, 0)),
            pl.BlockSpec(disen_weight_att.shape, lambda i: (0, 0)),
            pl.BlockSpec(weight.shape, lambda i: (0, 0)),
            pl.BlockSpec((BU, CH), lambda i: (i, 0)),
        ],
        out_specs=pl.BlockSpec((BU, CH), lambda i: (i, 0)),
        out_shape=jax.ShapeDtypeStruct((n_usr, CH), jnp.float32),
    )(user_emb, latent_emb, disen_weight_att, weight, user_agg)


def _packn(arrs, total_blocks):
    n = total_blocks
    m = jnp.stack([a.reshape(n, BLK) for a in arrs], axis=1)
    return m.reshape(-1)


def _pad_to(x, n, fill):
    pad = n - x.shape[0]
    return jnp.concatenate([x, jnp.full((pad,), fill, x.dtype)])


def kernel(entity_emb, user_emb, latent_emb, edge_index, edge_type,
           interact_rows, interact_cols, interact_vals, weight,
           disen_weight_att):
    head = edge_index[0].astype(jnp.int32)
    tail = edge_index[1].astype(jnp.int32)
    et = edge_type.astype(jnp.int32)
    ur = interact_rows.astype(jnp.int32)
    uc = interact_cols.astype(jnp.int32)

    unit = BLK * NSUB
    e1 = head.shape[0]
    e2 = ur.shape[0]
    nb1 = -(-e1 // unit)
    nb1 += nb1 % 2
    nb2 = -(-e2 // unit)
    nb2 += nb2 % 2
    tb1 = nb1 * NSUB + 1
    tb2 = nb2 * NSUB + 1

    kg_pack = _packn([
        _pad_to(tail, tb1 * BLK, 0),
        _pad_to(head, tb1 * BLK, GARBAGE),
        _pad_to(et, tb1 * BLK, 1) - 1,
    ], tb1)
    u_pack = _packn([
        _pad_to(uc, tb2 * BLK, 0),
        _pad_to(ur, tb2 * BLK, GARBAGE),
    ], tb2)
    u_vals = _pad_to(interact_vals, tb2 * BLK, 0.0)

    entity_agg, user_agg = _sc_aggregate(
        entity_emb, weight, kg_pack, u_pack, u_vals, nb1, nb2)
    n_ent = entity_emb.shape[0]
    entity_agg = entity_agg[:n_ent]
    user_agg = user_agg[:n_ent]

    user_out = _tc_modulate(user_emb, latent_emb, disen_weight_att, weight,
                            user_agg)
    return (entity_agg, user_out)


# speedup vs baseline: 4.8741x; 1.0702x over previous
import functools

import jax
import jax.numpy as jnp
from jax import lax
from jax.experimental import pallas as pl
from jax.experimental.pallas import tpu as pltpu
from jax.experimental.pallas import tpu_sc as plsc

CH = 128
LANES = 16
BLK = 128
PK = 3 * BLK
NSUB = 16
NCORE = 2
ACC_ROWS = 10240
GARBAGE = 10000


def _sc_body(ent_hbm, w_hbm, kg_pack, u_pack, u_vals, out_ent, out_usr,
             acc, cnt, ip0, ip1, sb0, sb1, rw0, rw1, wtab,
             vb0, vb1, ones_v, cb_v, zcnt_v, sg0, sg1, si0, si1,
             *, nb1, nb2):
    cid = lax.axis_index("c")
    sid = lax.axis_index("s")
    NB1 = nb1
    NB2 = nb2

    rows_per_tile = ACC_ROWS // NSUB

    def zero_row(r, _):
        zf = jnp.zeros((LANES,), jnp.float32)
        for k in range(CH // LANES):
            rw0[r, pl.ds(k * LANES, LANES)] = zf
        return 0

    lax.fori_loop(0, BLK, zero_row, 0)

    def zero_cnt1(g, _):
        zcnt_v[pl.ds(g * LANES, LANES)] = jnp.zeros((LANES,), jnp.float32)
        return 0

    lax.fori_loop(0, rows_per_tile // LANES, zero_cnt1, 0)

    def fill_ones(g, _):
        ones_v[pl.ds(g * LANES, LANES)] = jnp.full((LANES,), 1.0, jnp.float32)
        return 0

    lax.fori_loop(0, BLK // LANES, fill_ones, 0)

    for j in range(rows_per_tile // BLK):
        r0 = sid * rows_per_tile + j * BLK
        pltpu.sync_copy(rw0, acc.at[pl.ds(r0, BLK)])
    pltpu.sync_copy(zcnt_v, cnt.at[pl.ds(sid * rows_per_tile, rows_per_tile)])
    plsc.subcore_barrier()

    @pl.when(cid == 0)
    def _kg():
        pltpu.sync_copy(w_hbm, wtab)

        def pref(b, ip, si):
            base = (sid * NB1 + b) * PK
            pltpu.async_copy(kg_pack.at[pl.ds(base, PK)], ip, si)

        def pref_wait(ip, si):
            pltpu.make_async_copy(kg_pack.at[pl.ds(0, PK)], ip, si).wait()

        def issue(ip, sb, rw, si, sg):
            pref_wait(ip, si)
            for k in range(BLK // LANES):
                sb[pl.ds(k * LANES, LANES)] = ip[pl.ds(BLK + k * LANES, LANES)]
            pltpu.async_copy(ent_hbm.at[ip.at[pl.ds(0, BLK)]], rw, sg)

        def drain(ip, rw, sg):
            pltpu.make_async_copy(ent_hbm.at[ip.at[pl.ds(0, BLK)]], rw,
                                  sg).wait()

        def finish(bnext, ip, sb, rw, si, sg):
            drain(ip, rw, sg)

            def mul_group(g, _):
                tv = ip[pl.ds(2 * BLK + g * LANES, LANES)]
                for j in range(LANES):
                    e = g * LANES + j
                    t = tv[j]
                    for k in range(CH // LANES):
                        sl = pl.ds(k * LANES, LANES)
                        rw[e, sl] = rw[e, sl] * wtab[t, sl]
                return 0

            lax.fori_loop(0, BLK // LANES, mul_group, 0)
            pref(bnext, ip, si)
            pltpu.sync_copy(rw, acc.at[sb], add=True)
            pltpu.sync_copy(ones_v, cnt.at[sb], add=True)

        pref(0, ip0, si0)
        pref(1, ip1, si1)
        issue(ip0, sb0, rw0, si0, sg0)

        def pair(h, _):
            b0 = 2 * h
            issue(ip1, sb1, rw1, si1, sg1)
            finish(b0 + 2, ip0, sb0, rw0, si0, sg0)
            issue(ip0, sb0, rw0, si0, sg0)
            finish(b0 + 3, ip1, sb1, rw1, si1, sg1)
            return 0

        lax.fori_loop(0, NB1 // 2, pair, 0)
        drain(ip0, rw0, sg0)
        pref_wait(ip1, si1)

    @pl.when(cid == 1)
    def _usr():
        def pref(b, ip, vb, si):
            base = (sid * NB2 + b) * (2 * BLK)
            pltpu.async_copy(u_pack.at[pl.ds(base, 2 * BLK)],
                             ip.at[pl.ds(0, 2 * BLK)], si)
            vbase = (sid * NB2 + b) * BLK
            pltpu.async_copy(u_vals.at[pl.ds(vbase, BLK)], vb, si)

        def pref_wait(ip, vb, si):
            pltpu.make_async_copy(u_pack.at[pl.ds(0, 2 * BLK)],
                                  ip.at[pl.ds(0, 2 * BLK)], si).wait()
            pltpu.make_async_copy(u_vals.at[pl.ds(0, BLK)], vb, si).wait()

        def issue(ip, sb, rw, vb, si, sg):
            pref_wait(ip, vb, si)
            for k in range(BLK // LANES):
                sb[pl.ds(k * LANES, LANES)] = ip[pl.ds(BLK + k * LANES, LANES)]
            pltpu.async_copy(ent_hbm.at[ip.at[pl.ds(0, BLK)]], rw, sg)

        def drain(ip, rw, sg):
            pltpu.make_async_copy(ent_hbm.at[ip.at[pl.ds(0, BLK)]], rw,
                                  sg).wait()

        def finish(bnext, ip, sb, rw, vb, si, sg):
            drain(ip, rw, sg)

            def mul_group(g, _):
                vv = vb[pl.ds(g * LANES, LANES)]
                for j in range(LANES):
                    e = g * LANES + j
                    vj = vv[j]
                    for k in range(CH // LANES):
                        sl = pl.ds(k * LANES, LANES)
                        rw[e, sl] = rw[e, sl] * vj
                return 0

            lax.fori_loop(0, BLK // LANES, mul_group, 0)
            pref(bnext, ip, vb, si)
            pltpu.sync_copy(rw, acc.at[sb], add=True)

        pref(0, ip0, vb0, si0)
        pref(1, ip1, vb1, si1)
        issue(ip0, sb0, rw0, vb0, si0, sg0)

        def pair(h, _):
            b0 = 2 * h
            issue(ip1, sb1, rw1, vb1, si1, sg1)
            finish(b0 + 2, ip0, sb0, rw0, vb0, si0, sg0)
            issue(ip0, sb0, rw0, vb0, si0, sg0)
            finish(b0 + 3, ip1, sb1, rw1, vb1, si1, sg1)
            return 0

        lax.fori_loop(0, NB2 // 2, pair, 0)
        drain(ip0, rw0, sg0)
        pref_wait(ip1, vb1, si1)

    plsc.subcore_barrier()

    rpt = ACC_ROWS // NSUB

    @pl.when(cid == 0)
    def _wb_ent():
        def wchunk(j, _):
            r0 = sid * rpt + j * BLK
            pltpu.sync_copy(acc.at[pl.ds(r0, BLK)], rw0)
            pltpu.sync_copy(cnt.at[pl.ds(r0, BLK)], cb_v)

            def fix_group(g, _):
                cvec = cb_v[pl.ds(g * LANES, LANES)]
                inv = 1.0 / jnp.maximum(cvec, 1.0)
                for j2 in range(LANES):
                    r = g * LANES + j2
                    ij = inv[j2]
                    for k in range(CH // LANES):
                        sl = pl.ds(k * LANES, LANES)
                        rw0[r, sl] = rw0[r, sl] * ij
                return 0

            lax.fori_loop(0, BLK // LANES, fix_group, 0)
            pltpu.sync_copy(rw0, out_ent.at[pl.ds(r0, BLK)])
            return 0

        lax.fori_loop(0, rpt // BLK, wchunk, 0)

    @pl.when(cid == 1)
    def _wb_usr():
        def wchunk(j, _):
            r0 = sid * rpt + j * BLK
            pltpu.sync_copy(acc.at[pl.ds(r0, BLK)], rw0)
            pltpu.sync_copy(rw0, out_usr.at[pl.ds(r0, BLK)])
            return 0

        lax.fori_loop(0, rpt // BLK, wchunk, 0)


def _sc_aggregate(entity_emb, weight, kg_pack, u_pack, u_vals, nb1, nb2):
    mesh = plsc.VectorSubcoreMesh(core_axis_name="c", subcore_axis_name="s",
                                  num_cores=NCORE, num_subcores=NSUB)
    body = functools.partial(_sc_body, nb1=nb1, nb2=nb2)
    f = pl.kernel(
        body,
        out_type=(
            jax.ShapeDtypeStruct((ACC_ROWS, CH), jnp.float32),
            jax.ShapeDtypeStruct((ACC_ROWS, CH), jnp.float32),
        ),
        mesh=mesh,
        scratch_types=[
            pltpu.VMEM_SHARED((ACC_ROWS, CH), jnp.float32),
            pltpu.VMEM_SHARED((ACC_ROWS,), jnp.float32),
            pltpu.VMEM((PK,), jnp.int32),
            pltpu.VMEM((PK,), jnp.int32),
            pltpu.VMEM((BLK,), jnp.int32),
            pltpu.VMEM((BLK,), jnp.int32),
            pltpu.VMEM((BLK, CH), jnp.float32),
            pltpu.VMEM((BLK, CH), jnp.float32),
            pltpu.VMEM((16, CH), jnp.float32),
            pltpu.VMEM((BLK,), jnp.float32),
            pltpu.VMEM((BLK,), jnp.float32),
            pltpu.VMEM((BLK,), jnp.float32),
            pltpu.VMEM((BLK,), jnp.float32),
            pltpu.VMEM((ACC_ROWS // NSUB,), jnp.float32),
            pltpu.SemaphoreType.DMA,
            pltpu.SemaphoreType.DMA,
            pltpu.SemaphoreType.DMA,
            pltpu.SemaphoreType.DMA,
        ],
    )
    return f(entity_emb, weight, kg_pack, u_pack, u_vals)


def _tc_body(ue_ref, lat_ref, dis_ref, w_ref, ua_ref, out_ref):
    ue = ue_ref[...]
    s = lax.dot_general(ue, lat_ref[...],
                        (((1,), (1,)), ((), ())))
    s = jax.nn.softmax(s, axis=1)
    d = jax.nn.softmax(dis_ref[...], axis=-1) @ w_ref[...]
    m = 1.0 + s @ d
    out_ref[...] = ua_ref[...] * m


def _tc_modulate(user_emb, latent_emb, disen_weight_att, weight, user_agg):
    n_usr = user_emb.shape[0]
    BU = 1000
    grid = (n_usr // BU,)
    return pl.pallas_call(
        _tc_body,
        grid=grid,
        in_specs=[
            pl.BlockSpec((BU, CH), lambda i: (i, 0)),
            pl.BlockSpec(latent_emb.shape, lambda i: (0, 0)),
            pl.BlockSpec(disen_weight_att.shape, lambda i: (0, 0)),
            pl.BlockSpec(weight.shape, lambda i: (0, 0)),
            pl.BlockSpec((BU, CH), lambda i: (i, 0)),
        ],
        out_specs=pl.BlockSpec((BU, CH), lambda i: (i, 0)),
        out_shape=jax.ShapeDtypeStruct((n_usr, CH), jnp.float32),
    )(user_emb, latent_emb, disen_weight_att, weight, user_agg)


def _packn(arrs, total_blocks):
    n = total_blocks
    m = jnp.stack([a.reshape(n, BLK) for a in arrs], axis=1)
    return m.reshape(-1)


def _pad_to(x, n, fill):
    pad = n - x.shape[0]
    return jnp.concatenate([x, jnp.full((pad,), fill, x.dtype)])


def kernel(entity_emb, user_emb, latent_emb, edge_index, edge_type,
           interact_rows, interact_cols, interact_vals, weight,
           disen_weight_att):
    head = edge_index[0].astype(jnp.int32)
    tail = edge_index[1].astype(jnp.int32)
    et = edge_type.astype(jnp.int32)
    ur = interact_rows.astype(jnp.int32)
    uc = interact_cols.astype(jnp.int32)

    unit = BLK * NSUB
    e1 = head.shape[0]
    e2 = ur.shape[0]
    nb1 = -(-e1 // unit)
    nb1 += nb1 % 2
    nb2 = -(-e2 // unit)
    nb2 += nb2 % 2
    tb1 = nb1 * NSUB + 2
    tb2 = nb2 * NSUB + 2

    kg_pack = _packn([
        _pad_to(tail, tb1 * BLK, 0),
        _pad_to(head, tb1 * BLK, GARBAGE),
        _pad_to(et, tb1 * BLK, 1) - 1,
    ], tb1)
    u_pack = _packn([
        _pad_to(uc, tb2 * BLK, 0),
        _pad_to(ur, tb2 * BLK, GARBAGE),
    ], tb2)
    u_vals = _pad_to(interact_vals, tb2 * BLK, 0.0)

    entity_agg, user_agg = _sc_aggregate(
        entity_emb, weight, kg_pack, u_pack, u_vals, nb1, nb2)
    n_ent = entity_emb.shape[0]
    entity_agg = entity_agg[:n_ent]
    user_agg = user_agg[:n_ent]

    user_out = _tc_modulate(user_emb, latent_emb, disen_weight_att, weight,
                            user_agg)
    return (entity_agg, user_out)


# speedup vs baseline: 4.8911x; 1.0035x over previous
import functools

import jax
import jax.numpy as jnp
from jax import lax
from jax.experimental import pallas as pl
from jax.experimental.pallas import tpu as pltpu
from jax.experimental.pallas import tpu_sc as plsc

CH = 128
LANES = 16
BLK = 128
PK = 3 * BLK
NSUB = 16
NCORE = 2
ACC_ROWS = 10240
GARBAGE = 10000


def _sc_body(ent_hbm, w_hbm, kg_pack, u_pack, u_vals, out_ent, out_usr,
             acc, cnt, ip0, ip1, sb0, sb1, rw0, rw1, wtab,
             vb0, vb1, ones_v, cb_v, zcnt_v, sg0, sg1, si0, si1, ss0, ss1,
             *, nb1, nb2):
    cid = lax.axis_index("c")
    sid = lax.axis_index("s")
    NB1 = nb1
    NB2 = nb2

    rows_per_tile = ACC_ROWS // NSUB

    def zero_row(r, _):
        zf = jnp.zeros((LANES,), jnp.float32)
        for k in range(CH // LANES):
            rw0[r, pl.ds(k * LANES, LANES)] = zf
        return 0

    lax.fori_loop(0, BLK, zero_row, 0)

    def zero_cnt1(g, _):
        zcnt_v[pl.ds(g * LANES, LANES)] = jnp.zeros((LANES,), jnp.float32)
        return 0

    lax.fori_loop(0, rows_per_tile // LANES, zero_cnt1, 0)

    def fill_ones(g, _):
        ones_v[pl.ds(g * LANES, LANES)] = jnp.full((LANES,), 1.0, jnp.float32)
        return 0

    lax.fori_loop(0, BLK // LANES, fill_ones, 0)

    for j in range(rows_per_tile // BLK):
        r0 = sid * rows_per_tile + j * BLK
        pltpu.sync_copy(rw0, acc.at[pl.ds(r0, BLK)])
    pltpu.sync_copy(zcnt_v, cnt.at[pl.ds(sid * rows_per_tile, rows_per_tile)])
    plsc.subcore_barrier()

    @pl.when(cid == 0)
    def _kg():
        pltpu.sync_copy(w_hbm, wtab)

        def pref(b, ip, si):
            base = (sid * NB1 + b) * PK
            pltpu.async_copy(kg_pack.at[pl.ds(base, PK)], ip, si)

        def pref_wait(ip, si):
            pltpu.make_async_copy(kg_pack.at[pl.ds(0, PK)], ip, si).wait()

        def scat_wait(sb, rw, ss):
            pltpu.make_async_copy(rw, acc.at[sb], ss).wait()
            pltpu.make_async_copy(ones_v, cnt.at[sb], ss).wait()

        def issue(ip, sb, rw, si, sg, ss, first=False):
            pref_wait(ip, si)
            if not first:
                scat_wait(sb, rw, ss)
            for k in range(BLK // LANES):
                sb[pl.ds(k * LANES, LANES)] = ip[pl.ds(BLK + k * LANES, LANES)]
            pltpu.async_copy(ent_hbm.at[ip.at[pl.ds(0, BLK)]], rw, sg)

        def drain(ip, rw, sg):
            pltpu.make_async_copy(ent_hbm.at[ip.at[pl.ds(0, BLK)]], rw,
                                  sg).wait()

        def finish(bnext, ip, sb, rw, si, sg, ss):
            drain(ip, rw, sg)

            def mul_group(g, _):
                tv = ip[pl.ds(2 * BLK + g * LANES, LANES)]
                for j in range(LANES):
                    e = g * LANES + j
                    t = tv[j]
                    for k in range(CH // LANES):
                        sl = pl.ds(k * LANES, LANES)
                        rw[e, sl] = rw[e, sl] * wtab[t, sl]
                return 0

            lax.fori_loop(0, BLK // LANES, mul_group, 0)
            pref(bnext, ip, si)
            pltpu.async_copy(rw, acc.at[sb], ss, add=True)
            pltpu.async_copy(ones_v, cnt.at[sb], ss, add=True)

        pref(0, ip0, si0)
        pref(1, ip1, si1)
        issue(ip0, sb0, rw0, si0, sg0, ss0, first=True)

        def pair(h, _):
            b0 = 2 * h
            issue(ip1, sb1, rw1, si1, sg1, ss1)
            finish(b0 + 2, ip0, sb0, rw0, si0, sg0, ss0)
            issue(ip0, sb0, rw0, si0, sg0, ss0)
            finish(b0 + 3, ip1, sb1, rw1, si1, sg1, ss1)
            return 0

        issue(ip1, sb1, rw1, si1, sg1, ss1, first=True)
        finish(2, ip0, sb0, rw0, si0, sg0, ss0)
        issue(ip0, sb0, rw0, si0, sg0, ss0)
        finish(3, ip1, sb1, rw1, si1, sg1, ss1)
        lax.fori_loop(1, NB1 // 2, pair, 0)
        drain(ip0, rw0, sg0)
        scat_wait(sb1, rw1, ss1)
        pref_wait(ip1, si1)

    @pl.when(cid == 1)
    def _usr():
        def pref(b, ip, vb, si):
            base = (sid * NB2 + b) * (2 * BLK)
            pltpu.async_copy(u_pack.at[pl.ds(base, 2 * BLK)],
                             ip.at[pl.ds(0, 2 * BLK)], si)
            vbase = (sid * NB2 + b) * BLK
            pltpu.async_copy(u_vals.at[pl.ds(vbase, BLK)], vb, si)

        def pref_wait(ip, vb, si):
            pltpu.make_async_copy(u_pack.at[pl.ds(0, 2 * BLK)],
                                  ip.at[pl.ds(0, 2 * BLK)], si).wait()
            pltpu.make_async_copy(u_vals.at[pl.ds(0, BLK)], vb, si).wait()

        def scat_wait(sb, rw, ss):
            pltpu.make_async_copy(rw, acc.at[sb], ss).wait()

        def issue(ip, sb, rw, vb, si, sg, ss, first=False):
            pref_wait(ip, vb, si)
            if not first:
                scat_wait(sb, rw, ss)
            for k in range(BLK // LANES):
                sb[pl.ds(k * LANES, LANES)] = ip[pl.ds(BLK + k * LANES, LANES)]
            pltpu.async_copy(ent_hbm.at[ip.at[pl.ds(0, BLK)]], rw, sg)

        def drain(ip, rw, sg):
            pltpu.make_async_copy(ent_hbm.at[ip.at[pl.ds(0, BLK)]], rw,
                                  sg).wait()

        def finish(bnext, ip, sb, rw, vb, si, sg, ss):
            drain(ip, rw, sg)

            def mul_group(g, _):
                vv = vb[pl.ds(g * LANES, LANES)]
                for j in range(LANES):
                    e = g * LANES + j
                    vj = vv[j]
                    for k in range(CH // LANES):
                        sl = pl.ds(k * LANES, LANES)
                        rw[e, sl] = rw[e, sl] * vj
                return 0

            lax.fori_loop(0, BLK // LANES, mul_group, 0)
            pref(bnext, ip, vb, si)
            pltpu.async_copy(rw, acc.at[sb], ss, add=True)

        pref(0, ip0, vb0, si0)
        pref(1, ip1, vb1, si1)
        issue(ip0, sb0, rw0, vb0, si0, sg0, ss0, first=True)

        def pair(h, _):
            b0 = 2 * h
            issue(ip1, sb1, rw1, vb1, si1, sg1, ss1)
            finish(b0 + 2, ip0, sb0, rw0, vb0, si0, sg0, ss0)
            issue(ip0, sb0, rw0, vb0, si0, sg0, ss0)
            finish(b0 + 3, ip1, sb1, rw1, vb1, si1, sg1, ss1)
            return 0

        issue(ip1, sb1, rw1, vb1, si1, sg1, ss1, first=True)
        finish(2, ip0, sb0, rw0, vb0, si0, sg0, ss0)
        issue(ip0, sb0, rw0, vb0, si0, sg0, ss0)
        finish(3, ip1, sb1, rw1, vb1, si1, sg1, ss1)
        lax.fori_loop(1, NB2 // 2, pair, 0)
        drain(ip0, rw0, sg0)
        scat_wait(sb1, rw1, ss1)
        pref_wait(ip1, vb1, si1)

    plsc.subcore_barrier()

    rpt = ACC_ROWS // NSUB

    @pl.when(cid == 0)
    def _wb_ent():
        def wchunk(j, _):
            r0 = sid * rpt + j * BLK
            pltpu.sync_copy(acc.at[pl.ds(r0, BLK)], rw0)
            pltpu.sync_copy(cnt.at[pl.ds(r0, BLK)], cb_v)

            def fix_group(g, _):
                cvec = cb_v[pl.ds(g * LANES, LANES)]
                inv = 1.0 / jnp.maximum(cvec, 1.0)
                for j2 in range(LANES):
                    r = g * LANES + j2
                    ij = inv[j2]
                    for k in range(CH // LANES):
                        sl = pl.ds(k * LANES, LANES)
                        rw0[r, sl] = rw0[r, sl] * ij
                return 0

            lax.fori_loop(0, BLK // LANES, fix_group, 0)
            pltpu.sync_copy(rw0, out_ent.at[pl.ds(r0, BLK)])
            return 0

        lax.fori_loop(0, rpt // BLK, wchunk, 0)

    @pl.when(cid == 1)
    def _wb_usr():
        def wchunk(j, _):
            r0 = sid * rpt + j * BLK
            pltpu.sync_copy(acc.at[pl.ds(r0, BLK)], rw0)
            pltpu.sync_copy(rw0, out_usr.at[pl.ds(r0, BLK)])
            return 0

        lax.fori_loop(0, rpt // BLK, wchunk, 0)


def _sc_aggregate(entity_emb, weight, kg_pack, u_pack, u_vals, nb1, nb2):
    mesh = plsc.VectorSubcoreMesh(core_axis_name="c", subcore_axis_name="s",
                                  num_cores=NCORE, num_subcores=NSUB)
    body = functools.partial(_sc_body, nb1=nb1, nb2=nb2)
    f = pl.kernel(
        body,
        out_type=(
            jax.ShapeDtypeStruct((ACC_ROWS, CH), jnp.float32),
            jax.ShapeDtypeStruct((ACC_ROWS, CH), jnp.float32),
        ),
        mesh=mesh,
        scratch_types=[
            pltpu.VMEM_SHARED((ACC_ROWS, CH), jnp.float32),
            pltpu.VMEM_SHARED((ACC_ROWS,), jnp.float32),
            pltpu.VMEM((PK,), jnp.int32),
            pltpu.VMEM((PK,), jnp.int32),
            pltpu.VMEM((BLK,), jnp.int32),
            pltpu.VMEM((BLK,), jnp.int32),
            pltpu.VMEM((BLK, CH), jnp.float32),
            pltpu.VMEM((BLK, CH), jnp.float32),
            pltpu.VMEM((16, CH), jnp.float32),
            pltpu.VMEM((BLK,), jnp.float32),
            pltpu.VMEM((BLK,), jnp.float32),
            pltpu.VMEM((BLK,), jnp.float32),
            pltpu.VMEM((BLK,), jnp.float32),
            pltpu.VMEM((ACC_ROWS // NSUB,), jnp.float32),
            pltpu.SemaphoreType.DMA,
            pltpu.SemaphoreType.DMA,
            pltpu.SemaphoreType.DMA,
            pltpu.SemaphoreType.DMA,
            pltpu.SemaphoreType.DMA,
            pltpu.SemaphoreType.DMA,
        ],
    )
    return f(entity_emb, weight, kg_pack, u_pack, u_vals)


def _tc_body(ue_ref, lat_ref, dis_ref, w_ref, ua_ref, out_ref):
    ue = ue_ref[...]
    s = lax.dot_general(ue, lat_ref[...],
                        (((1,), (1,)), ((), ())))
    s = jax.nn.softmax(s, axis=1)
    d = jax.nn.softmax(dis_ref[...], axis=-1) @ w_ref[...]
    m = 1.0 + s @ d
    out_ref[...] = ua_ref[...] * m


def _tc_modulate(user_emb, latent_emb, disen_weight_att, weight, user_agg):
    n_usr = user_emb.shape[0]
    BU = 1000
    grid = (n_usr // BU,)
    return pl.pallas_call(
        _tc_body,
        grid=grid,
        in_specs=[
            pl.BlockSpec((BU, CH), lambda i: (i, 0)),
            pl.BlockSpec(latent_emb.shape, lambda i: (0, 0)),
            pl.BlockSpec(disen_weight_att.shape, lambda i: (0, 0)),
            pl.BlockSpec(weight.shape, lambda i: (0, 0)),
            pl.BlockSpec((BU, CH), lambda i: (i, 0)),
        ],
        out_specs=pl.BlockSpec((BU, CH), lambda i: (i, 0)),
        out_shape=jax.ShapeDtypeStruct((n_usr, CH), jnp.float32),
    )(user_emb, latent_emb, disen_weight_att, weight, user_agg)


def _packn(arrs, total_blocks):
    n = total_blocks
    m = jnp.stack([a.reshape(n, BLK) for a in arrs], axis=1)
    return m.reshape(-1)


def _pad_to(x, n, fill):
    pad = n - x.shape[0]
    return jnp.concatenate([x, jnp.full((pad,), fill, x.dtype)])


def kernel(entity_emb, user_emb, latent_emb, edge_index, edge_type,
           interact_rows, interact_cols, interact_vals, weight,
           disen_weight_att):
    head = edge_index[0].astype(jnp.int32)
    tail = edge_index[1].astype(jnp.int32)
    et = edge_type.astype(jnp.int32)
    ur = interact_rows.astype(jnp.int32)
    uc = interact_cols.astype(jnp.int32)

    unit = BLK * NSUB
    e1 = head.shape[0]
    e2 = ur.shape[0]
    nb1 = -(-e1 // unit)
    nb1 += nb1 % 2
    nb2 = -(-e2 // unit)
    nb2 += nb2 % 2
    tb1 = nb1 * NSUB + 2
    tb2 = nb2 * NSUB + 2

    kg_pack = _packn([
        _pad_to(tail, tb1 * BLK, 0),
        _pad_to(head, tb1 * BLK, GARBAGE),
        _pad_to(et, tb1 * BLK, 1) - 1,
    ], tb1)
    u_pack = _packn([
        _pad_to(uc, tb2 * BLK, 0),
        _pad_to(ur, tb2 * BLK, GARBAGE),
    ], tb2)
    u_vals = _pad_to(interact_vals, tb2 * BLK, 0.0)

    entity_agg, user_agg = _sc_aggregate(
        entity_emb, weight, kg_pack, u_pack, u_vals, nb1, nb2)
    n_ent = entity_emb.shape[0]
    entity_agg = entity_agg[:n_ent]
    user_agg = user_agg[:n_ent]

    user_out = _tc_modulate(user_emb, latent_emb, disen_weight_att, weight,
                            user_agg)
    return (entity_agg, user_out)
